# baseline (speedup 1.0000x reference)
import functools

import jax
import jax.numpy as jnp
from jax.experimental import pallas as pl


def _moe_body(x_ref, wg_ref, we_ref, bet_ref, out_ref, gate_ref, *, nb, e, out_len):
    b = pl.program_id(0)
    xb = x_ref[0]
    nv = xb.shape[1]

    logits = jnp.dot(wg_ref[...], xb, preferred_element_type=jnp.float32)
    m = jnp.max(logits, axis=0, keepdims=True)
    ex = jnp.exp(logits - m)
    g = ex / jnp.sum(ex, axis=0, keepdims=True)

    iota = jax.lax.broadcasted_iota(jnp.int32, (e, nv), 0)
    m1 = jnp.max(g, axis=0, keepdims=True)
    idx1 = jnp.min(jnp.where(g == m1, iota, e), axis=0, keepdims=True)
    mask1 = iota == idx1
    neg = jnp.float32(-jnp.inf)
    g2 = jnp.where(mask1, neg, g)
    m2 = jnp.max(g2, axis=0, keepdims=True)
    idx2 = jnp.min(jnp.where(g2 == m2, iota, e), axis=0, keepdims=True)
    mask2 = iota == idx2
    wd = jnp.where(mask1 | mask2, g, 0.0)

    y = jnp.dot(
        we_ref[...].astype(jnp.bfloat16),
        xb.astype(jnp.bfloat16),
        preferred_element_type=jnp.float32,
    )
    acc = jnp.dot(bet_ref[...], wd, preferred_element_type=jnp.float32)
    for i in range(e):
        acc = acc + wd[i : i + 1, :] * y[i * out_len : (i + 1) * out_len, :]
    out_ref[0] = acc

    @pl.when(b == 0)
    def _init():
        gate_ref[...] = jnp.zeros_like(gate_ref)

    gate_ref[...] += g

    @pl.when(b == nb - 1)
    def _fin():
        gate_ref[...] = gate_ref[...] * (1.0 / nb)


@jax.jit
def kernel(x, Wg, We, be):
    B, IN_LEN, NVARS = x.shape
    E, OUT_LEN, _ = We.shape

    we_flat = We.reshape(E * OUT_LEN, IN_LEN)
    be_t = be.T

    body = functools.partial(_moe_body, nb=B, e=E, out_len=OUT_LEN)
    out, gate_sum = pl.pallas_call(
        body,
        grid=(B,),
        in_specs=[
            pl.BlockSpec((1, IN_LEN, NVARS), lambda b: (b, 0, 0)),
            pl.BlockSpec((E, IN_LEN), lambda b: (0, 0)),
            pl.BlockSpec((E * OUT_LEN, IN_LEN), lambda b: (0, 0)),
            pl.BlockSpec((OUT_LEN, E), lambda b: (0, 0)),
        ],
        out_specs=[
            pl.BlockSpec((1, OUT_LEN, NVARS), lambda b: (b, 0, 0)),
            pl.BlockSpec((E, NVARS), lambda b: (0, 0)),
        ],
        out_shape=[
            jax.ShapeDtypeStruct((B, OUT_LEN, NVARS), x.dtype),
            jax.ShapeDtypeStruct((E, NVARS), jnp.float32),
        ],
    )(x, Wg, we_flat, be_t)

    gate_mean = gate_sum.T
    return (out, gate_mean)


# speedup vs baseline: 1.1385x; 1.1385x over previous
import functools

import jax
import jax.numpy as jnp
from jax.experimental import pallas as pl


def _moe_body(x_ref, wg_ref, we_ref, bet_ref, out_ref, gate_ref, *, nb, e, out_len):
    b = pl.program_id(0)
    xb = x_ref[0]
    nv = xb.shape[1]

    logits = jnp.dot(wg_ref[...], xb, preferred_element_type=jnp.float32)
    m = jnp.max(logits, axis=0, keepdims=True)
    ex = jnp.exp(logits - m)
    g = ex / jnp.sum(ex, axis=0, keepdims=True)

    iota = jax.lax.broadcasted_iota(jnp.int32, (e, nv), 0)
    m1 = jnp.max(g, axis=0, keepdims=True)
    idx1 = jnp.min(jnp.where(g == m1, iota, e), axis=0, keepdims=True)
    mask1 = iota == idx1
    neg = jnp.float32(-jnp.inf)
    g2 = jnp.where(mask1, neg, g)
    m2 = jnp.max(g2, axis=0, keepdims=True)
    idx2 = jnp.min(jnp.where(g2 == m2, iota, e), axis=0, keepdims=True)
    mask2 = iota == idx2
    wd = jnp.where(mask1 | mask2, g, 0.0)

    acc = jnp.dot(bet_ref[...], wd, preferred_element_type=jnp.float32)
    for i in range(e):
        acc = acc + jnp.dot(
            we_ref[i], wd[i : i + 1, :] * xb, preferred_element_type=jnp.float32
        )
    out_ref[0] = acc

    @pl.when(b == 0)
    def _init():
        gate_ref[...] = jnp.zeros_like(gate_ref)

    gate_ref[...] += g

    @pl.when(b == nb - 1)
    def _fin():
        gate_ref[...] = gate_ref[...] * (1.0 / nb)


@jax.jit
def kernel(x, Wg, We, be):
    B, IN_LEN, NVARS = x.shape
    E, OUT_LEN, _ = We.shape

    be_t = be.T

    body = functools.partial(_moe_body, nb=B, e=E, out_len=OUT_LEN)
    out, gate_sum = pl.pallas_call(
        body,
        grid=(B,),
        in_specs=[
            pl.BlockSpec((1, IN_LEN, NVARS), lambda b: (b, 0, 0)),
            pl.BlockSpec((E, IN_LEN), lambda b: (0, 0)),
            pl.BlockSpec((E, OUT_LEN, IN_LEN), lambda b: (0, 0, 0)),
            pl.BlockSpec((OUT_LEN, E), lambda b: (0, 0)),
        ],
        out_specs=[
            pl.BlockSpec((1, OUT_LEN, NVARS), lambda b: (b, 0, 0)),
            pl.BlockSpec((E, NVARS), lambda b: (0, 0)),
        ],
        out_shape=[
            jax.ShapeDtypeStruct((B, OUT_LEN, NVARS), x.dtype),
            jax.ShapeDtypeStruct((E, NVARS), jnp.float32),
        ],
    )(x, Wg, We, be_t)

    gate_mean = gate_sum.T
    return (out, gate_mean)


# speedup vs baseline: 1.2512x; 1.0990x over previous
import functools

import jax
import jax.numpy as jnp
from jax.experimental import pallas as pl


def _moe_body(x_ref, wg_ref, we_ref, bet_ref, out_ref, gate_ref, *, nb, e, out_len):
    b = pl.program_id(0)
    xb = x_ref[0]
    nv = xb.shape[1]

    logits = jnp.dot(wg_ref[...], xb, preferred_element_type=jnp.float32)
    m = jnp.max(logits, axis=0, keepdims=True)
    ex = jnp.exp(logits - m)
    g = ex / jnp.sum(ex, axis=0, keepdims=True)

    iota = jax.lax.broadcasted_iota(jnp.int32, (e, nv), 0)
    m1 = jnp.max(g, axis=0, keepdims=True)
    idx1 = jnp.min(jnp.where(g == m1, iota, e), axis=0, keepdims=True)
    mask1 = iota == idx1
    neg = jnp.float32(-jnp.inf)
    g2 = jnp.where(mask1, neg, g)
    m2 = jnp.max(g2, axis=0, keepdims=True)
    idx2 = jnp.min(jnp.where(g2 == m2, iota, e), axis=0, keepdims=True)
    mask2 = iota == idx2
    wd = jnp.where(mask1 | mask2, g, 0.0)

    xw = jnp.concatenate(
        [wd[i : i + 1, :] * xb for i in range(e)], axis=0
    ).astype(jnp.bfloat16)
    acc = jnp.dot(bet_ref[...], wd, preferred_element_type=jnp.float32)
    acc = acc + jnp.dot(we_ref[...], xw, preferred_element_type=jnp.float32)
    out_ref[0] = acc

    @pl.when(b == 0)
    def _init():
        gate_ref[...] = jnp.zeros_like(gate_ref)

    gate_ref[...] += g

    @pl.when(b == nb - 1)
    def _fin():
        gate_ref[...] = gate_ref[...] * (1.0 / nb)


@jax.jit
def kernel(x, Wg, We, be):
    B, IN_LEN, NVARS = x.shape
    E, OUT_LEN, _ = We.shape

    be_t = be.T
    we_cat = We.transpose(1, 0, 2).reshape(OUT_LEN, E * IN_LEN).astype(jnp.bfloat16)

    body = functools.partial(_moe_body, nb=B, e=E, out_len=OUT_LEN)
    out, gate_sum = pl.pallas_call(
        body,
        grid=(B,),
        in_specs=[
            pl.BlockSpec((1, IN_LEN, NVARS), lambda b: (b, 0, 0)),
            pl.BlockSpec((E, IN_LEN), lambda b: (0, 0)),
            pl.BlockSpec((OUT_LEN, E * IN_LEN), lambda b: (0, 0)),
            pl.BlockSpec((OUT_LEN, E), lambda b: (0, 0)),
        ],
        out_specs=[
            pl.BlockSpec((1, OUT_LEN, NVARS), lambda b: (b, 0, 0)),
            pl.BlockSpec((E, NVARS), lambda b: (0, 0)),
        ],
        out_shape=[
            jax.ShapeDtypeStruct((B, OUT_LEN, NVARS), x.dtype),
            jax.ShapeDtypeStruct((E, NVARS), jnp.float32),
        ],
    )(x, Wg, we_cat, be_t)

    gate_mean = gate_sum.T
    return (out, gate_mean)


# speedup vs baseline: 1.2788x; 1.0221x over previous
import functools

import jax
import jax.numpy as jnp
from jax.experimental import pallas as pl


def _moe_body(x_ref, wg_ref, we_ref, out_ref, gate_ref, *, nb, e, out_len):
    b = pl.program_id(0)
    xb = x_ref[0]
    nv = xb.shape[1]

    logits = jnp.dot(wg_ref[...], xb, preferred_element_type=jnp.float32)
    m = jnp.max(logits, axis=0, keepdims=True)
    ex = jnp.exp(logits - m)
    g = ex / jnp.sum(ex, axis=0, keepdims=True)

    iota = jax.lax.broadcasted_iota(jnp.int32, (e, nv), 0)
    m1 = jnp.max(g, axis=0, keepdims=True)
    idx1 = jnp.min(jnp.where(g == m1, iota, e), axis=0, keepdims=True)
    mask1 = iota == idx1
    neg = jnp.float32(-jnp.inf)
    g2 = jnp.where(mask1, neg, g)
    m2 = jnp.max(g2, axis=0, keepdims=True)
    idx2 = jnp.min(jnp.where(g2 == m2, iota, e), axis=0, keepdims=True)
    mask2 = iota == idx2
    wd = jnp.where(mask1 | mask2, g, 0.0)

    xb_b = xb.astype(jnp.bfloat16)
    wd_b = wd.astype(jnp.bfloat16)
    xw = jnp.concatenate(
        [wd_b[i : i + 1, :] * xb_b for i in range(e)] + [wd_b], axis=0
    )
    out_ref[0] = jnp.dot(we_ref[...], xw, preferred_element_type=jnp.float32)

    @pl.when(b == 0)
    def _init():
        gate_ref[...] = jnp.zeros_like(gate_ref)

    gate_ref[...] += g

    @pl.when(b == nb - 1)
    def _fin():
        gate_ref[...] = gate_ref[...] * (1.0 / nb)


@jax.jit
def kernel(x, Wg, We, be):
    B, IN_LEN, NVARS = x.shape
    E, OUT_LEN, _ = We.shape

    we_cat = jnp.concatenate(
        [We.transpose(1, 0, 2).reshape(OUT_LEN, E * IN_LEN), be.T], axis=1
    ).astype(jnp.bfloat16)

    body = functools.partial(_moe_body, nb=B, e=E, out_len=OUT_LEN)
    out, gate_sum = pl.pallas_call(
        body,
        grid=(B,),
        in_specs=[
            pl.BlockSpec((1, IN_LEN, NVARS), lambda b: (b, 0, 0)),
            pl.BlockSpec((E, IN_LEN), lambda b: (0, 0)),
            pl.BlockSpec((OUT_LEN, E * IN_LEN + E), lambda b: (0, 0)),
        ],
        out_specs=[
            pl.BlockSpec((1, OUT_LEN, NVARS), lambda b: (b, 0, 0)),
            pl.BlockSpec((E, NVARS), lambda b: (0, 0)),
        ],
        out_shape=[
            jax.ShapeDtypeStruct((B, OUT_LEN, NVARS), x.dtype),
            jax.ShapeDtypeStruct((E, NVARS), jnp.float32),
        ],
    )(x, Wg, we_cat)

    gate_mean = gate_sum.T
    return (out, gate_mean)


# speedup vs baseline: 1.6142x; 1.2623x over previous
import functools

import jax
import jax.numpy as jnp
from jax.experimental import pallas as pl


def _moe_slice(xb, wg, we_cat, e):
    nv = xb.shape[1]

    logits = jnp.dot(wg, xb, preferred_element_type=jnp.float32)
    m = jnp.max(logits, axis=0, keepdims=True)
    ex = jnp.exp(logits - m)
    g = ex / jnp.sum(ex, axis=0, keepdims=True)

    iota = jax.lax.broadcasted_iota(jnp.int32, (e, nv), 0)
    m1 = jnp.max(g, axis=0, keepdims=True)
    idx1 = jnp.min(jnp.where(g == m1, iota, e), axis=0, keepdims=True)
    mask1 = iota == idx1
    neg = jnp.float32(-jnp.inf)
    g2 = jnp.where(mask1, neg, g)
    m2 = jnp.max(g2, axis=0, keepdims=True)
    idx2 = jnp.min(jnp.where(g2 == m2, iota, e), axis=0, keepdims=True)
    mask2 = iota == idx2
    wd = jnp.where(mask1 | mask2, g, 0.0)

    xb_b = xb.astype(jnp.bfloat16)
    wd_b = wd.astype(jnp.bfloat16)
    xw = jnp.concatenate(
        [wd_b[i : i + 1, :] * xb_b for i in range(e)] + [wd_b], axis=0
    )
    out = jnp.dot(we_cat, xw, preferred_element_type=jnp.float32)
    return out, g


def _moe_body(x_ref, wg_ref, we_ref, out_ref, gate_ref, *, nsteps, bt, e):
    s = pl.program_id(0)
    wg = wg_ref[...]
    we_cat = we_ref[...]

    gsum = None
    for bi in range(bt):
        out, g = _moe_slice(x_ref[bi], wg, we_cat, e)
        out_ref[bi] = out
        gsum = g if gsum is None else gsum + g

    @pl.when(s == 0)
    def _init():
        gate_ref[...] = jnp.zeros_like(gate_ref)

    gate_ref[...] += gsum

    @pl.when(s == nsteps - 1)
    def _fin():
        gate_ref[...] = gate_ref[...] * (1.0 / (nsteps * bt))


@jax.jit
def kernel(x, Wg, We, be):
    B, IN_LEN, NVARS = x.shape
    E, OUT_LEN, _ = We.shape
    BT = 2
    nsteps = B // BT

    we_cat = jnp.concatenate(
        [We.transpose(1, 0, 2).reshape(OUT_LEN, E * IN_LEN), be.T], axis=1
    ).astype(jnp.bfloat16)

    body = functools.partial(_moe_body, nsteps=nsteps, bt=BT, e=E)
    out, gate_sum = pl.pallas_call(
        body,
        grid=(nsteps,),
        in_specs=[
            pl.BlockSpec((BT, IN_LEN, NVARS), lambda s: (s, 0, 0)),
            pl.BlockSpec((E, IN_LEN), lambda s: (0, 0)),
            pl.BlockSpec((OUT_LEN, E * IN_LEN + E), lambda s: (0, 0)),
        ],
        out_specs=[
            pl.BlockSpec((BT, OUT_LEN, NVARS), lambda s: (s, 0, 0)),
            pl.BlockSpec((E, NVARS), lambda s: (0, 0)),
        ],
        out_shape=[
            jax.ShapeDtypeStruct((B, OUT_LEN, NVARS), x.dtype),
            jax.ShapeDtypeStruct((E, NVARS), jnp.float32),
        ],
    )(x, Wg, we_cat)

    gate_mean = gate_sum.T
    return (out, gate_mean)


# speedup vs baseline: 1.7564x; 1.0881x over previous
import functools

import jax
import jax.numpy as jnp
from jax.experimental import pallas as pl


def _moe_slice(xb, wg, we_cat, e):
    nv = xb.shape[1]

    logits = jnp.dot(wg, xb, preferred_element_type=jnp.float32)
    m = jnp.max(logits, axis=0, keepdims=True)
    ex = jnp.exp(logits - m)
    g = ex / jnp.sum(ex, axis=0, keepdims=True)

    iota = jax.lax.broadcasted_iota(jnp.int32, (e, nv), 0)
    m1 = jnp.max(g, axis=0, keepdims=True)
    idx1 = jnp.min(jnp.where(g == m1, iota, e), axis=0, keepdims=True)
    mask1 = iota == idx1
    neg = jnp.float32(-jnp.inf)
    g2 = jnp.where(mask1, neg, g)
    m2 = jnp.max(g2, axis=0, keepdims=True)
    idx2 = jnp.min(jnp.where(g2 == m2, iota, e), axis=0, keepdims=True)
    mask2 = iota == idx2
    wd = jnp.where(mask1 | mask2, g, 0.0)

    xb_b = xb.astype(jnp.bfloat16)
    wd_b = wd.astype(jnp.bfloat16)
    xw = jnp.concatenate(
        [wd_b[i : i + 1, :] * xb_b for i in range(e)] + [wd_b], axis=0
    )
    out = jnp.dot(we_cat, xw, preferred_element_type=jnp.float32)
    return out, g


def _moe_body(x_ref, wg_ref, we_ref, out_ref, gate_ref, *, nsteps, bt, e):
    s = pl.program_id(0)
    wg = wg_ref[...]
    we_cat = we_ref[...]

    gsum = None
    for bi in range(bt):
        out, g = _moe_slice(x_ref[bi], wg, we_cat, e)
        out_ref[bi] = out
        gsum = g if gsum is None else gsum + g

    @pl.when(s == 0)
    def _init():
        gate_ref[...] = jnp.zeros_like(gate_ref)

    gate_ref[...] += gsum

    @pl.when(s == nsteps - 1)
    def _fin():
        gate_ref[...] = gate_ref[...] * (1.0 / (nsteps * bt))


@jax.jit
def kernel(x, Wg, We, be):
    B, IN_LEN, NVARS = x.shape
    E, OUT_LEN, _ = We.shape
    BT = 8
    nsteps = B // BT

    we_cat = jnp.concatenate(
        [We.transpose(1, 0, 2).reshape(OUT_LEN, E * IN_LEN), be.T], axis=1
    ).astype(jnp.bfloat16)

    body = functools.partial(_moe_body, nsteps=nsteps, bt=BT, e=E)
    out, gate_sum = pl.pallas_call(
        body,
        grid=(nsteps,),
        in_specs=[
            pl.BlockSpec((BT, IN_LEN, NVARS), lambda s: (s, 0, 0)),
            pl.BlockSpec((E, IN_LEN), lambda s: (0, 0)),
            pl.BlockSpec((OUT_LEN, E * IN_LEN + E), lambda s: (0, 0)),
        ],
        out_specs=[
            pl.BlockSpec((BT, OUT_LEN, NVARS), lambda s: (s, 0, 0)),
            pl.BlockSpec((E, NVARS), lambda s: (0, 0)),
        ],
        out_shape=[
            jax.ShapeDtypeStruct((B, OUT_LEN, NVARS), x.dtype),
            jax.ShapeDtypeStruct((E, NVARS), jnp.float32),
        ],
    )(x, Wg, we_cat)

    gate_mean = gate_sum.T
    return (out, gate_mean)
